# baseline (speedup 1.0000x reference)
import functools
import jax
import jax.numpy as jnp
from jax import lax
from jax.experimental import pallas as pl
from jax.experimental.pallas import tpu as pltpu
from jax.experimental.pallas import tpu_sc as plsc

H, W = 480, 640
HW = H * W
PLANE = 2 * HW
NC, NS = 2, 16
CH = 1024
NCHUNK = 16
PER_TILE = CH * NCHUNK
N_PAD = NS * PER_TILE
STRIPE = PLANE // NS
LIN_BLK = N_PAD // 16


def _tc_linearize(B, N):
    nblk = N_PAD // LIN_BLK

    def body(y_i, x_i, p_i, t_i, y_o, x_o, p_o, t_o):
        i = pl.program_id(0)
        b = pl.program_id(1)
        pos = i * LIN_BLK + lax.broadcasted_iota(jnp.int32, (LIN_BLK,), 0)
        m = pos < N
        yv, xv, pv, tv = lax.switch(
            b,
            [lambda bb=bb: (y_i[bb, :], x_i[bb, :], p_i[bb, :], t_i[bb, :])
             for bb in range(B)])
        y_o[...] = jnp.where(m, yv, -5.0)
        x_o[...] = jnp.where(m, xv, -5.0)
        p_o[...] = jnp.where(m, pv, 0.0)
        t_o[...] = jnp.where(m, tv, 0.0)

    f = pl.pallas_call(
        body,
        grid=(nblk, B),
        in_specs=[pl.BlockSpec((B, LIN_BLK), lambda i, b: (0, i))] * 4,
        out_specs=[pl.BlockSpec((LIN_BLK,), lambda i, b: (b * nblk + i,))] * 4,
        out_shape=[jax.ShapeDtypeStruct((B * N_PAD,), jnp.float32)] * 4,
    )
    return f


def _make_sc_kernel(B):
    assert B % NC == 0
    BPC = B // NC
    G = CH // 16

    mesh = plsc.VectorSubcoreMesh(core_axis_name="c", subcore_axis_name="s")

    @functools.partial(
        pl.kernel,
        out_type=[
            jax.ShapeDtypeStruct((B * PLANE,), jnp.float32),
            jax.ShapeDtypeStruct((B * PLANE,), jnp.float32),
        ],
        mesh=mesh,
        compiler_params=pltpu.CompilerParams(needs_layout_passes=False),
        scratch_types=[
            pltpu.VMEM((CH,), jnp.float32),
            pltpu.VMEM((CH,), jnp.float32),
            pltpu.VMEM((CH,), jnp.float32),
            pltpu.VMEM((CH,), jnp.float32),
            pltpu.VMEM((CH,), jnp.float32),
            pltpu.VMEM((CH,), jnp.float32),
            pltpu.VMEM((CH,), jnp.float32),
            pltpu.VMEM((CH,), jnp.float32),
            pltpu.VMEM((4 * CH,), jnp.int32),
            pltpu.VMEM((4 * CH,), jnp.int32),
            pltpu.VMEM((4 * CH,), jnp.float32),
            pltpu.VMEM((4 * CH,), jnp.float32),
            pltpu.VMEM((4 * CH,), jnp.float32),
            pltpu.VMEM((4 * CH,), jnp.float32),
            pltpu.VMEM((16,), jnp.float32),
            pltpu.VMEM((16,), jnp.float32),
            pltpu.VMEM_SHARED((PLANE,), jnp.float32),
            pltpu.VMEM_SHARED((PLANE,), jnp.float32),
            pltpu.SemaphoreType.DMA,
            pltpu.SemaphoreType.DMA,
            pltpu.SemaphoreType.DMA,
            pltpu.SemaphoreType.DMA,
        ],
    )
    def k(y_hbm, x_hbm, p_hbm, t_hbm, tref_hbm, inv_hbm, zeros_hbm,
          out_w, out_t,
          y0, y1, x0, x1, p0, p1, t0, t1,
          idx0, idx1, w0, w1, wt0, wt1,
          tref_v, inv_v, acc_w, acc_t, semi0, semi1, sems0, sems1):
        c = lax.axis_index("c")
        s = lax.axis_index("s")

        pltpu.sync_copy(tref_hbm, tref_v)
        pltpu.sync_copy(inv_hbm, inv_v)
        tref = tref_v[...]
        inv = inv_v[...]

        in_bufs = [(y0, x0, p0, t0, semi0), (y1, x1, p1, t1, semi1)]
        sc_bufs = [(idx0, w0, wt0, sems0), (idx1, w1, wt1, sems1)]

        def fire_in(b, j, bi):
            ys, xs, ps, ts_, sem = in_bufs[bi]
            off = b * N_PAD + s * PER_TILE + j * CH
            col = pl.ds(off, CH)
            return [
                pltpu.async_copy(y_hbm.at[col], ys, sem),
                pltpu.async_copy(x_hbm.at[col], xs, sem),
                pltpu.async_copy(p_hbm.at[col], ps, sem),
                pltpu.async_copy(t_hbm.at[col], ts_, sem),
            ]

        def do_groups(bi):
            ys, xs, ps, ts_, _ = in_bufs[bi]
            idx_v, w_v, wt_v, _ = sc_bufs[bi]

            def group(g, _):
                o16 = g * 16
                y = ys[pl.ds(o16, 16)]
                x = xs[pl.ds(o16, 16)]
                p = ps[pl.ds(o16, 16)]
                t = ts_[pl.ds(o16, 16)]
                iy = y.astype(jnp.int32)
                ix = x.astype(jnp.int32)
                fy = y - iy.astype(jnp.float32)
                fx = x - ix.astype(jnp.float32)
                nt = 1.0 - jnp.abs(tref - t) * inv
                chan = 1 - p.astype(jnp.int32)
                valid = ((iy >= 0) & (iy <= H - 2) & (ix >= 0) & (ix <= W - 2))
                base = jnp.where(valid, chan * HW + iy * W + ix, 0)
                wy0 = 1.0 - fy
                wx0 = 1.0 - fx
                w00 = jnp.where(valid, wy0 * wx0, 0.0)
                w01 = jnp.where(valid, wy0 * fx, 0.0)
                w10 = jnp.where(valid, fy * wx0, 0.0)
                w11 = jnp.where(valid, fy * fx, 0.0)
                o = g * 64
                idx_v[pl.ds(o, 16)] = base
                idx_v[pl.ds(o + 16, 16)] = base + 1
                idx_v[pl.ds(o + 32, 16)] = base + W
                idx_v[pl.ds(o + 48, 16)] = base + W + 1
                w_v[pl.ds(o, 16)] = w00
                w_v[pl.ds(o + 16, 16)] = w01
                w_v[pl.ds(o + 32, 16)] = w10
                w_v[pl.ds(o + 48, 16)] = w11
                wt_v[pl.ds(o, 16)] = w00 * nt
                wt_v[pl.ds(o + 16, 16)] = w01 * nt
                wt_v[pl.ds(o + 32, 16)] = w10 * nt
                wt_v[pl.ds(o + 48, 16)] = w11 * nt
                return 0

            lax.fori_loop(0, G, group, 0)

        for bi in range(BPC):
            b = c * BPC + bi
            pltpu.sync_copy(zeros_hbm, acc_w.at[pl.ds(s * STRIPE, STRIPE)])
            pltpu.sync_copy(zeros_hbm, acc_t.at[pl.ds(s * STRIPE, STRIPE)])
            pend_in = [fire_in(b, 0, 0), None]
            plsc.subcore_barrier()

            pend_sc = [None, None]
            for j in range(NCHUNK):
                pb = j % 2
                if j + 1 < NCHUNK:
                    pend_in[1 - pb] = fire_in(b, j + 1, 1 - pb)
                for d in pend_in[pb]:
                    d.wait()
                if pend_sc[pb] is not None:
                    for d in pend_sc[pb]:
                        d.wait()
                do_groups(pb)
                idx_v, w_v, wt_v, sem = sc_bufs[pb]
                pend_sc[pb] = [
                    pltpu.async_copy(w_v, acc_w.at[idx_v], sem, add=True),
                    pltpu.async_copy(wt_v, acc_t.at[idx_v], sem, add=True),
                ]
            for pb in (0, 1):
                for d in pend_sc[pb]:
                    d.wait()

            plsc.subcore_barrier()
            pltpu.sync_copy(acc_w.at[pl.ds(s * STRIPE, STRIPE)],
                            out_w.at[pl.ds(b * PLANE + s * STRIPE, STRIPE)])
            pltpu.sync_copy(acc_t.at[pl.ds(s * STRIPE, STRIPE)],
                            out_t.at[pl.ds(b * PLANE + s * STRIPE, STRIPE)])

    return k


def kernel(warped_events, pol_mask, ts_list, tref, ts_scaling):
    B, N, _ = warped_events.shape
    y2 = warped_events[:, :, 0]
    x2 = warped_events[:, :, 1]
    p2 = pol_mask[:, :N, 0]
    t2 = ts_list[:, :, 0]
    y1, x1, p1, t1 = _tc_linearize(B, N)(y2, x2, p2, t2)
    tref16 = jnp.full((16,), tref[0], dtype=jnp.float32)
    inv16 = jnp.full((16,), 1.0 / ts_scaling[0], dtype=jnp.float32)
    zeros = jnp.zeros((STRIPE,), dtype=jnp.float32)
    out_w, out_t = _make_sc_kernel(B)(y1, x1, p1, t1, tref16, inv16, zeros)
    return (out_w.reshape(B, 2, H, W), out_t.reshape(B, 2, H, W))


# speedup vs baseline: 1.0337x; 1.0337x over previous
import functools
import jax
import jax.numpy as jnp
from jax import lax
from jax.experimental import pallas as pl
from jax.experimental.pallas import tpu as pltpu
from jax.experimental.pallas import tpu_sc as plsc

H, W = 480, 640
HW = H * W
PLANE = 2 * HW
NC, NS = 2, 16
CH = 1024
NCHUNK = 16
PER_TILE = CH * NCHUNK
N_PAD = NS * PER_TILE
STRIPE = PLANE // NS
LIN_BLK = N_PAD // 16


def _tc_linearize(B, N):
    nblk = N_PAD // LIN_BLK

    def body(y_i, x_i, p_i, t_i, y_o, x_o, pt_o):
        i = pl.program_id(0)
        b = pl.program_id(1)
        pos = i * LIN_BLK + lax.broadcasted_iota(jnp.int32, (LIN_BLK,), 0)
        m = pos < N
        y_o[...] = jnp.where(m, y_i[b, :], -5.0)
        x_o[...] = jnp.where(m, x_i[b, :], -5.0)
        pt = (t_i[b, :] + 2.0) * (p_i[b, :] * 2.0 - 1.0)
        pt_o[...] = jnp.where(m, pt, 1.0)

    f = pl.pallas_call(
        body,
        grid=(nblk, B),
        in_specs=[pl.BlockSpec((B, LIN_BLK), lambda i, b: (0, i))] * 4,
        out_specs=[pl.BlockSpec((LIN_BLK,), lambda i, b: (b * nblk + i,))] * 3,
        out_shape=[jax.ShapeDtypeStruct((B * N_PAD,), jnp.float32)] * 3,
    )
    return f


def _make_sc_kernel(B):
    assert B % NC == 0
    BPC = B // NC
    G = CH // 16

    mesh = plsc.VectorSubcoreMesh(core_axis_name="c", subcore_axis_name="s")

    @functools.partial(
        pl.kernel,
        out_type=[
            jax.ShapeDtypeStruct((B * PLANE,), jnp.float32),
            jax.ShapeDtypeStruct((B * PLANE,), jnp.float32),
        ],
        mesh=mesh,
        compiler_params=pltpu.CompilerParams(needs_layout_passes=False),
        scratch_types=[
            pltpu.VMEM((CH,), jnp.float32),
            pltpu.VMEM((CH,), jnp.float32),
            pltpu.VMEM((CH,), jnp.float32),
            pltpu.VMEM((CH,), jnp.float32),
            pltpu.VMEM((CH,), jnp.float32),
            pltpu.VMEM((CH,), jnp.float32),
            pltpu.VMEM((4 * CH,), jnp.int32),
            pltpu.VMEM((4 * CH,), jnp.int32),
            pltpu.VMEM((4 * CH,), jnp.float32),
            pltpu.VMEM((4 * CH,), jnp.float32),
            pltpu.VMEM((4 * CH,), jnp.float32),
            pltpu.VMEM((4 * CH,), jnp.float32),
            pltpu.VMEM((16,), jnp.float32),
            pltpu.VMEM((16,), jnp.float32),
            pltpu.VMEM_SHARED((PLANE,), jnp.float32),
            pltpu.VMEM_SHARED((PLANE,), jnp.float32),
            pltpu.SemaphoreType.DMA,
            pltpu.SemaphoreType.DMA,
            pltpu.SemaphoreType.DMA,
            pltpu.SemaphoreType.DMA,
        ],
    )
    def k(y_hbm, x_hbm, pt_hbm, tref_hbm, inv_hbm, zeros_hbm,
          out_w, out_t,
          y0, y1, x0, x1, pt0, pt1,
          idx0, idx1, w0, w1, wt0, wt1,
          tref_v, inv_v, acc_w, acc_t, semi0, semi1, sems0, sems1):
        c = lax.axis_index("c")
        s = lax.axis_index("s")

        pltpu.sync_copy(tref_hbm, tref_v)
        pltpu.sync_copy(inv_hbm, inv_v)
        tref = tref_v[...]
        inv = inv_v[...]

        in_bufs = [(y0, x0, pt0, semi0), (y1, x1, pt1, semi1)]
        sc_bufs = [(idx0, w0, wt0, sems0), (idx1, w1, wt1, sems1)]

        def fire_in(b, j, bi):
            ys, xs, pts, sem = in_bufs[bi]
            off = b * N_PAD + s * PER_TILE + j * CH
            col = pl.ds(off, CH)
            return [
                pltpu.async_copy(y_hbm.at[col], ys, sem),
                pltpu.async_copy(x_hbm.at[col], xs, sem),
                pltpu.async_copy(pt_hbm.at[col], pts, sem),
            ]

        def do_groups(bi):
            ys, xs, pts, _ = in_bufs[bi]
            idx_v, w_v, wt_v, _ = sc_bufs[bi]

            def group(g, _):
                o16 = g * 16
                y = ys[pl.ds(o16, 16)]
                x = xs[pl.ds(o16, 16)]
                pt = pts[pl.ds(o16, 16)]
                iy = y.astype(jnp.int32)
                ix = x.astype(jnp.int32)
                fy = y - iy.astype(jnp.float32)
                fx = x - ix.astype(jnp.float32)
                t = jnp.abs(pt) - 2.0
                nt = 1.0 - jnp.abs(tref - t) * inv
                chan = jnp.where(pt > 0, 0, 1)
                valid = ((iy >= 0) & (iy <= H - 2) & (ix >= 0) & (ix <= W - 2))
                base = jnp.where(valid, chan * HW + iy * W + ix, 0)
                wy0 = 1.0 - fy
                wx0 = 1.0 - fx
                w00 = jnp.where(valid, wy0 * wx0, 0.0)
                w01 = jnp.where(valid, wy0 * fx, 0.0)
                w10 = jnp.where(valid, fy * wx0, 0.0)
                w11 = jnp.where(valid, fy * fx, 0.0)
                o = g * 64
                idx_v[pl.ds(o, 16)] = base
                idx_v[pl.ds(o + 16, 16)] = base + 1
                idx_v[pl.ds(o + 32, 16)] = base + W
                idx_v[pl.ds(o + 48, 16)] = base + W + 1
                w_v[pl.ds(o, 16)] = w00
                w_v[pl.ds(o + 16, 16)] = w01
                w_v[pl.ds(o + 32, 16)] = w10
                w_v[pl.ds(o + 48, 16)] = w11
                wt_v[pl.ds(o, 16)] = w00 * nt
                wt_v[pl.ds(o + 16, 16)] = w01 * nt
                wt_v[pl.ds(o + 32, 16)] = w10 * nt
                wt_v[pl.ds(o + 48, 16)] = w11 * nt
                return 0

            lax.fori_loop(0, G, group, 0)

        for bi in range(BPC):
            b = c * BPC + bi
            pltpu.sync_copy(zeros_hbm, acc_w.at[pl.ds(s * STRIPE, STRIPE)])
            pltpu.sync_copy(zeros_hbm, acc_t.at[pl.ds(s * STRIPE, STRIPE)])
            pend_in = [fire_in(b, 0, 0), None]
            plsc.subcore_barrier()

            pend_sc = [None, None]
            for j in range(NCHUNK):
                pb = j % 2
                if j + 1 < NCHUNK:
                    pend_in[1 - pb] = fire_in(b, j + 1, 1 - pb)
                for d in pend_in[pb]:
                    d.wait()
                if pend_sc[pb] is not None:
                    for d in pend_sc[pb]:
                        d.wait()
                do_groups(pb)
                idx_v, w_v, wt_v, sem = sc_bufs[pb]
                pend_sc[pb] = [
                    pltpu.async_copy(w_v, acc_w.at[idx_v], sem, add=True),
                    pltpu.async_copy(wt_v, acc_t.at[idx_v], sem, add=True),
                ]
            for pb in (0, 1):
                for d in pend_sc[pb]:
                    d.wait()

            plsc.subcore_barrier()
            pltpu.sync_copy(acc_w.at[pl.ds(s * STRIPE, STRIPE)],
                            out_w.at[pl.ds(b * PLANE + s * STRIPE, STRIPE)])
            pltpu.sync_copy(acc_t.at[pl.ds(s * STRIPE, STRIPE)],
                            out_t.at[pl.ds(b * PLANE + s * STRIPE, STRIPE)])

    return k


def kernel(warped_events, pol_mask, ts_list, tref, ts_scaling):
    B, N, _ = warped_events.shape
    y2 = warped_events[:, :, 0]
    x2 = warped_events[:, :, 1]
    p2 = pol_mask[:, :N, 0]
    t2 = ts_list[:, :, 0]
    y1, x1, pt1 = _tc_linearize(B, N)(y2, x2, p2, t2)
    tref16 = jnp.full((16,), tref[0], dtype=jnp.float32)
    inv16 = jnp.full((16,), 1.0 / ts_scaling[0], dtype=jnp.float32)
    zeros = jnp.zeros((STRIPE,), dtype=jnp.float32)
    out_w, out_t = _make_sc_kernel(B)(y1, x1, pt1, tref16, inv16, zeros)
    return (out_w.reshape(B, 2, H, W), out_t.reshape(B, 2, H, W))


# speedup vs baseline: 1.0366x; 1.0028x over previous
import functools
import jax
import jax.numpy as jnp
from jax import lax
from jax.experimental import pallas as pl
from jax.experimental.pallas import tpu as pltpu
from jax.experimental.pallas import tpu_sc as plsc

H, W = 480, 640
HW = H * W
PLANE = 2 * HW
NC, NS = 2, 16
CH = 1024
NCHUNK = 16
PER_TILE = CH * NCHUNK
N_PAD = NS * PER_TILE
STRIPE = PLANE // NS
LIN_BLK = N_PAD // 16


def _tc_linearize(B, N):
    nblk = N_PAD // LIN_BLK

    def body(y_i, x_i, p_i, t_i, y_o, x_o, pt_o):
        i = pl.program_id(0)
        b = pl.program_id(1)
        pos = i * LIN_BLK + lax.broadcasted_iota(jnp.int32, (LIN_BLK,), 0)
        m = pos < N
        y_o[...] = jnp.where(m, y_i[b, :], -5.0)
        x_o[...] = jnp.where(m, x_i[b, :], -5.0)
        pt = (t_i[b, :] + 2.0) * (p_i[b, :] * 2.0 - 1.0)
        pt_o[...] = jnp.where(m, pt, 1.0)

    f = pl.pallas_call(
        body,
        grid=(nblk, B),
        in_specs=[pl.BlockSpec((B, LIN_BLK), lambda i, b: (0, i))] * 4,
        out_specs=[pl.BlockSpec((LIN_BLK,), lambda i, b: (b * nblk + i,))] * 3,
        out_shape=[jax.ShapeDtypeStruct((B * N_PAD,), jnp.float32)] * 3,
    )
    return f


def _make_sc_kernel(B):
    assert B % NC == 0
    BPC = B // NC
    G = CH // 16

    mesh = plsc.VectorSubcoreMesh(core_axis_name="c", subcore_axis_name="s")

    @functools.partial(
        pl.kernel,
        out_type=[
            jax.ShapeDtypeStruct((B * PLANE,), jnp.float32),
            jax.ShapeDtypeStruct((B * PLANE,), jnp.float32),
        ],
        mesh=mesh,
        compiler_params=pltpu.CompilerParams(needs_layout_passes=False),
        scratch_types=[
            pltpu.VMEM((CH,), jnp.float32),
            pltpu.VMEM((CH,), jnp.float32),
            pltpu.VMEM((CH,), jnp.float32),
            pltpu.VMEM((CH,), jnp.float32),
            pltpu.VMEM((CH,), jnp.float32),
            pltpu.VMEM((CH,), jnp.float32),
            pltpu.VMEM((4 * CH,), jnp.int32),
            pltpu.VMEM((4 * CH,), jnp.int32),
            pltpu.VMEM((4 * CH,), jnp.float32),
            pltpu.VMEM((4 * CH,), jnp.float32),
            pltpu.VMEM((4 * CH,), jnp.float32),
            pltpu.VMEM((4 * CH,), jnp.float32),
            pltpu.VMEM((16,), jnp.float32),
            pltpu.VMEM((16,), jnp.float32),
            pltpu.VMEM_SHARED((PLANE,), jnp.float32),
            pltpu.VMEM_SHARED((PLANE,), jnp.float32),
            pltpu.SemaphoreType.DMA,
            pltpu.SemaphoreType.DMA,
            pltpu.SemaphoreType.DMA,
            pltpu.SemaphoreType.DMA,
        ],
    )
    def k(y_hbm, x_hbm, pt_hbm, tref_hbm, inv_hbm, zeros_hbm,
          out_w, out_t,
          y0, y1, x0, x1, pt0, pt1,
          idx0, idx1, w0, w1, wt0, wt1,
          tref_v, inv_v, acc_w, acc_t, semi0, semi1, sems0, sems1):
        c = lax.axis_index("c")
        s = lax.axis_index("s")

        pltpu.sync_copy(tref_hbm, tref_v)
        pltpu.sync_copy(inv_hbm, inv_v)
        tref = tref_v[...]
        inv = inv_v[...]

        in_bufs = [(y0, x0, pt0, semi0), (y1, x1, pt1, semi1)]
        sc_bufs = [(idx0, w0, wt0, sems0), (idx1, w1, wt1, sems1)]

        def fire_in(b, j, bi):
            ys, xs, pts, sem = in_bufs[bi]
            off = b * N_PAD + s * PER_TILE + j * CH
            col = pl.ds(off, CH)
            return [
                pltpu.async_copy(y_hbm.at[col], ys, sem),
                pltpu.async_copy(x_hbm.at[col], xs, sem),
                pltpu.async_copy(pt_hbm.at[col], pts, sem),
            ]

        def do_groups(bi):
            ys, xs, pts, _ = in_bufs[bi]
            idx_v, w_v, wt_v, _ = sc_bufs[bi]

            def group(g, _):
                o16 = g * 16
                y = ys[pl.ds(o16, 16)]
                x = xs[pl.ds(o16, 16)]
                pt = pts[pl.ds(o16, 16)]
                iy = y.astype(jnp.int32)
                ix = x.astype(jnp.int32)
                fy = y - iy.astype(jnp.float32)
                fx = x - ix.astype(jnp.float32)
                t = jnp.abs(pt) - 2.0
                nt = 1.0 - jnp.abs(tref - t) * inv
                chan = jnp.where(pt > 0, 0, 1)
                valid = ((iy >= 0) & (iy <= H - 2) & (ix >= 0) & (ix <= W - 2))
                base = jnp.where(valid, chan * HW + iy * W + ix, 0)
                wy0 = 1.0 - fy
                wx0 = 1.0 - fx
                w00 = jnp.where(valid, wy0 * wx0, 0.0)
                w01 = jnp.where(valid, wy0 * fx, 0.0)
                w10 = jnp.where(valid, fy * wx0, 0.0)
                w11 = jnp.where(valid, fy * fx, 0.0)
                o = g * 64
                idx_v[pl.ds(o, 16)] = base
                idx_v[pl.ds(o + 16, 16)] = base + 1
                idx_v[pl.ds(o + 32, 16)] = base + W
                idx_v[pl.ds(o + 48, 16)] = base + W + 1
                w_v[pl.ds(o, 16)] = w00
                w_v[pl.ds(o + 16, 16)] = w01
                w_v[pl.ds(o + 32, 16)] = w10
                w_v[pl.ds(o + 48, 16)] = w11
                wt_v[pl.ds(o, 16)] = w00 * nt
                wt_v[pl.ds(o + 16, 16)] = w01 * nt
                wt_v[pl.ds(o + 32, 16)] = w10 * nt
                wt_v[pl.ds(o + 48, 16)] = w11 * nt
                return 0

            lax.fori_loop(0, G, group, 0)

        for bi in range(BPC):
            b = c * BPC + bi
            pltpu.sync_copy(zeros_hbm, acc_w.at[pl.ds(s * STRIPE, STRIPE)])
            pltpu.sync_copy(zeros_hbm, acc_t.at[pl.ds(s * STRIPE, STRIPE)])
            pend_in = [fire_in(b, 0, 0), None]
            plsc.subcore_barrier()

            pend_sc = [None, None]
            for j in range(NCHUNK):
                pb = j % 2
                if j + 1 < NCHUNK:
                    pend_in[1 - pb] = fire_in(b, j + 1, 1 - pb)
                for d in pend_in[pb]:
                    d.wait()
                if pend_sc[pb] is not None:
                    for d in pend_sc[pb]:
                        d.wait()
                do_groups(pb)
                idx_v, w_v, wt_v, sem = sc_bufs[pb]
                pend_sc[pb] = [
                    pltpu.async_copy(w_v, acc_w.at[idx_v], sem, add=True),
                    pltpu.async_copy(wt_v, acc_t.at[idx_v], sem, add=True),
                ]
            for pb in (0, 1):
                for d in pend_sc[pb]:
                    d.wait()

            plsc.subcore_barrier()
            pltpu.sync_copy(acc_w.at[pl.ds(s * STRIPE, STRIPE)],
                            out_w.at[pl.ds(b * PLANE + s * STRIPE, STRIPE)])
            pltpu.sync_copy(acc_t.at[pl.ds(s * STRIPE, STRIPE)],
                            out_t.at[pl.ds(b * PLANE + s * STRIPE, STRIPE)])

    return k


def kernel(warped_events, pol_mask, ts_list, tref, ts_scaling):
    B, N, _ = warped_events.shape
    y2 = warped_events[:, :, 0]
    x2 = warped_events[:, :, 1]
    p2 = pol_mask[:, :N, 0]
    t2 = ts_list[:, :, 0]
    tref16 = jnp.full((16,), tref[0], dtype=jnp.float32)
    inv16 = jnp.full((16,), 1.0 / ts_scaling[0], dtype=jnp.float32)
    zeros = jnp.zeros((STRIPE,), dtype=jnp.float32)
    HB = B // 2
    lin = _tc_linearize(HB, N)
    sck = _make_sc_kernel(HB)
    outs = []
    for h in range(2):
        sl = slice(h * HB, (h + 1) * HB)
        y1, x1, pt1 = lin(y2[sl], x2[sl], p2[sl], t2[sl])
        outs.append(sck(y1, x1, pt1, tref16, inv16, zeros))
    out_w = jnp.concatenate([outs[0][0], outs[1][0]])
    out_t = jnp.concatenate([outs[0][1], outs[1][1]])
    return (out_w.reshape(B, 2, H, W), out_t.reshape(B, 2, H, W))
